# Initial kernel scaffold; baseline (speedup 1.0000x reference)
import jax
import jax.numpy as jnp
from jax.experimental import pallas as pl


def kernel(pitch_type_id, x_bin, z_bin, description_id, cont, emb_pitch_type, emb_x, emb_z, emb_desc, W1, b1, W2, b2, cont_mean, cont_std):
    raise NotImplementedError("write your pallas kernel here")


# speedup vs baseline: 5.4254x; 5.4254x over previous
import functools

import jax
import jax.numpy as jnp
from jax.experimental import pallas as pl
from jax.experimental.pallas import tpu as pltpu


def _round_up(x, m):
    return (x + m - 1) // m * m


def _fold_kernel(ept_ref, ex_ref, ez_ref, ed_ref, w1_ref,
                 a1_ref, a2_ref, *, dims):
    (n_pt, d_pt, n_x, d_x, n_z, d_z, n_d, d_d, n_c, L1, L2, TB, D) = dims
    w1 = w1_ref[...]
    r0 = d_pt
    r1 = d_pt + d_x
    r2 = d_pt + d_x + d_z
    r3 = d_pt + d_x + d_z + d_d
    a_pt = jnp.dot(ept_ref[...], w1[0:r0, :],
                   preferred_element_type=jnp.float32)
    a_x = jnp.dot(ex_ref[...], w1[r0:r1, :],
                  preferred_element_type=jnp.float32)
    a_z = jnp.dot(ez_ref[...], w1[r1:r2, :],
                  preferred_element_type=jnp.float32)
    a_d = jnp.dot(ed_ref[...], w1[r2:r3, :],
                  preferred_element_type=jnp.float32)
    a_c = w1[r3:r3 + n_c, :]
    a1_ref[...] = jnp.concatenate([a_pt, a_c], axis=0)
    pad2 = jnp.zeros((L2 - n_x - n_z - n_d, D), jnp.float32)
    a2_ref[...] = jnp.concatenate([a_x, a_z, a_d, pad2], axis=0)


def _encoder_kernel(pt_ref, x_ref, z_ref, d_ref, c1_ref, a1_ref, a2_ref,
                    b1_ref, w2_ref, b2_ref, mean_ref, istd_ref,
                    out_ref, *, dims):
    (n_pt, d_pt, n_x, d_x, n_z, d_z, n_d, d_d, n_c, L1, L2, TB, D) = dims

    iota1 = jax.lax.broadcasted_iota(jnp.int32, (TB, L1), 1)
    aug1 = (iota1 == pt_ref[...]).astype(jnp.float32)
    aug1 += (c1_ref[...] - mean_ref[...]) * istd_ref[...]

    iota2 = jax.lax.broadcasted_iota(jnp.int32, (TB, L2), 1)
    aug2 = (iota2 == x_ref[...]).astype(jnp.float32)
    aug2 += (iota2 == z_ref[...] + n_x).astype(jnp.float32)
    aug2 += (iota2 == d_ref[...] + n_x + n_z).astype(jnp.float32)

    h = jnp.dot(aug1, a1_ref[...], preferred_element_type=jnp.float32)
    h += jnp.dot(aug2, a2_ref[...], preferred_element_type=jnp.float32)
    h += b1_ref[...]
    h = 0.5 * h * (1.0 + jax.lax.erf(h * 0.7071067811865476))
    out = jnp.dot(h, w2_ref[...], preferred_element_type=jnp.float32)
    out_ref[...] = out + b2_ref[...]


@jax.jit
def kernel(pitch_type_id, x_bin, z_bin, description_id, cont,
           emb_pitch_type, emb_x, emb_z, emb_desc,
           W1, b1, W2, b2, cont_mean, cont_std):
    B, S = pitch_type_id.shape
    T = B * S
    n_pt, d_pt = emb_pitch_type.shape
    n_x, d_x = emb_x.shape
    n_z, d_z = emb_z.shape
    n_d, d_d = emb_desc.shape
    n_c = cont.shape[-1]
    D = W2.shape[1]
    L1 = n_pt + n_c
    L2 = _round_up(n_x + n_z + n_d, 8)

    TB = 1024
    assert T % TB == 0
    G = T // TB

    dims = (n_pt, d_pt, n_x, d_x, n_z, d_z, n_d, d_d, n_c, L1, L2, TB, D)
    full = lambda shape: pl.BlockSpec(shape, lambda i: (0, 0))

    a1, a2 = pl.pallas_call(
        functools.partial(_fold_kernel, dims=dims),
        out_shape=(jax.ShapeDtypeStruct((L1, D), jnp.float32),
                   jax.ShapeDtypeStruct((L2, D), jnp.float32)),
    )(emb_pitch_type, emb_x, emb_z, emb_desc, W1)

    c1 = jnp.pad(cont.reshape(T, n_c), ((0, 0), (n_pt, 0)))
    mean_pad = jnp.pad(cont_mean.reshape(1, n_c), ((0, 0), (n_pt, 0)))
    istd_pad = jnp.pad(1.0 / jnp.clip(cont_std, 1e-6, None).reshape(1, n_c),
                       ((0, 0), (n_pt, 0)))

    idcol = lambda a: a.reshape(T, 1).astype(jnp.int32)

    out = pl.pallas_call(
        functools.partial(_encoder_kernel, dims=dims),
        grid=(G,),
        in_specs=[
            pl.BlockSpec((TB, 1), lambda i: (i, 0)),
            pl.BlockSpec((TB, 1), lambda i: (i, 0)),
            pl.BlockSpec((TB, 1), lambda i: (i, 0)),
            pl.BlockSpec((TB, 1), lambda i: (i, 0)),
            pl.BlockSpec((TB, L1), lambda i: (i, 0)),
            full((L1, D)),
            full((L2, D)),
            full((1, D)),
            full((D, D)),
            full((1, D)),
            full((1, L1)),
            full((1, L1)),
        ],
        out_specs=pl.BlockSpec((TB, D), lambda i: (i, 0)),
        out_shape=jax.ShapeDtypeStruct((T, D), jnp.float32),
        compiler_params=pltpu.CompilerParams(
            dimension_semantics=("parallel",)),
    )(idcol(pitch_type_id), idcol(x_bin), idcol(z_bin),
      idcol(description_id), c1, a1, a2,
      b1.reshape(1, D), W2, b2.reshape(1, D), mean_pad, istd_pad)

    return out.reshape(B, S, D)


# speedup vs baseline: 9.7656x; 1.8000x over previous
import functools

import jax
import jax.numpy as jnp
from jax.experimental import pallas as pl
from jax.experimental.pallas import tpu as pltpu


def _round_up(x, m):
    return (x + m - 1) // m * m


def _fold_kernel(ept_ref, ex_ref, ez_ref, ed_ref, w1_ref, istd_ref,
                 a1_ref, a2_ref, *, dims):
    (n_pt, d_pt, n_x, d_x, n_z, d_z, n_d, d_d, n_c, L1, L2, TB, D) = dims
    w1 = w1_ref[...]
    r0 = d_pt
    r1 = d_pt + d_x
    r2 = d_pt + d_x + d_z
    r3 = d_pt + d_x + d_z + d_d
    a_pt = jnp.dot(ept_ref[...], w1[0:r0, :],
                   preferred_element_type=jnp.float32)
    a_x = jnp.dot(ex_ref[...], w1[r0:r1, :],
                  preferred_element_type=jnp.float32)
    a_z = jnp.dot(ez_ref[...], w1[r1:r2, :],
                  preferred_element_type=jnp.float32)
    a_d = jnp.dot(ed_ref[...], w1[r2:r3, :],
                  preferred_element_type=jnp.float32)
    a_c = w1[r3:r3 + n_c, :] * istd_ref[...]
    z4 = jnp.zeros((n_c, D), jnp.float32)
    a1_ref[...] = jnp.concatenate(
        [z4, a_c, a_pt, jnp.zeros((L1 - 2 * n_c - n_pt, D), jnp.float32)],
        axis=0)
    pad2 = jnp.zeros((L2 - n_x - n_z - n_d, D), jnp.float32)
    a2_ref[...] = jnp.concatenate([a_x, a_z, a_d, pad2], axis=0)


def _encoder_kernel(pk_ref, a1_ref, a2_ref, b1_ref, w2_ref, b2_ref,
                    moff_ref, out_ref, *, dims):
    (n_pt, d_pt, n_x, d_x, n_z, d_z, n_d, d_d, n_c, L1, L2, TB, D) = dims

    pk = pk_ref[...]
    iota1 = jax.lax.broadcasted_iota(jnp.int32, (TB, L1), 1)
    pkm = pk * moff_ref[0:1, 0:8] + moff_ref[1:2, 0:8]
    aug1 = jnp.pad(pkm, ((0, 0), (0, L1 - 2 * n_c)))
    aug1 += (iota1 == pk[:, 0:1].astype(jnp.int32) + 2 * n_c).astype(
        jnp.float32)

    iota2 = jax.lax.broadcasted_iota(jnp.int32, (TB, L2), 1)
    ix = pk[:, 1:2].astype(jnp.int32)
    iz = pk[:, 2:3].astype(jnp.int32)
    id_ = pk[:, 3:4].astype(jnp.int32)
    aug2 = (iota2 == ix).astype(jnp.float32)
    aug2 += (iota2 == iz + n_x).astype(jnp.float32)
    aug2 += (iota2 == id_ + n_x + n_z).astype(jnp.float32)

    h = jnp.dot(aug1, a1_ref[...], preferred_element_type=jnp.float32)
    h += jnp.dot(aug2, a2_ref[...], preferred_element_type=jnp.float32)
    h += b1_ref[...]
    h = 0.5 * h * (1.0 + jax.lax.erf(h * 0.7071067811865476))
    out = jnp.dot(h, w2_ref[...], preferred_element_type=jnp.float32)
    out_ref[...] = out + b2_ref[...]


@jax.jit
def kernel(pitch_type_id, x_bin, z_bin, description_id, cont,
           emb_pitch_type, emb_x, emb_z, emb_desc,
           W1, b1, W2, b2, cont_mean, cont_std):
    B, S = pitch_type_id.shape
    T = B * S
    n_pt, d_pt = emb_pitch_type.shape
    n_x, d_x = emb_x.shape
    n_z, d_z = emb_z.shape
    n_d, d_d = emb_desc.shape
    n_c = cont.shape[-1]
    D = W2.shape[1]
    L1 = _round_up(2 * n_c + n_pt, 8)
    L2 = _round_up(n_x + n_z + n_d, 8)

    TB = 1024
    assert T % TB == 0
    G = T // TB

    istd = 1.0 / jnp.clip(cont_std, 1e-6, None)

    packed = jnp.concatenate(
        [jnp.stack([pitch_type_id, x_bin, z_bin, description_id],
                   axis=-1).astype(jnp.float32),
         cont.reshape(T, n_c).reshape(B, S, n_c)],
        axis=-1).reshape(T, 2 * n_c)

    mrow = jnp.concatenate([jnp.zeros((n_c,)), jnp.ones((n_c,))])
    orow = jnp.concatenate([jnp.zeros((n_c,)), -cont_mean * istd])
    moff = jnp.stack([mrow, orow]).astype(jnp.float32)

    dims = (n_pt, d_pt, n_x, d_x, n_z, d_z, n_d, d_d, n_c, L1, L2, TB, D)
    full = lambda shape: pl.BlockSpec(shape, lambda i: (0, 0))

    a1, a2 = pl.pallas_call(
        functools.partial(_fold_kernel, dims=dims),
        out_shape=(jax.ShapeDtypeStruct((L1, D), jnp.float32),
                   jax.ShapeDtypeStruct((L2, D), jnp.float32)),
    )(emb_pitch_type, emb_x, emb_z, emb_desc, W1, istd.reshape(n_c, 1))

    out = pl.pallas_call(
        functools.partial(_encoder_kernel, dims=dims),
        grid=(G,),
        in_specs=[
            pl.BlockSpec((TB, 2 * n_c), lambda i: (i, 0)),
            full((L1, D)),
            full((L2, D)),
            full((1, D)),
            full((D, D)),
            full((1, D)),
            full((2, 2 * n_c)),
        ],
        out_specs=pl.BlockSpec((TB, D), lambda i: (i, 0)),
        out_shape=jax.ShapeDtypeStruct((T, D), jnp.float32),
        compiler_params=pltpu.CompilerParams(
            dimension_semantics=("parallel",)),
    )(packed, a1, a2, b1.reshape(1, D), W2, b2.reshape(1, D), moff)

    return out.reshape(B, S, D)


# speedup vs baseline: 10.1841x; 1.0429x over previous
import functools

import jax
import jax.numpy as jnp
from jax.experimental import pallas as pl
from jax.experimental.pallas import tpu as pltpu


def _round_up(x, m):
    return (x + m - 1) // m * m


def _fold_kernel(ept_ref, ex_ref, ez_ref, ed_ref, w1_ref, istd_ref,
                 a1_ref, a2_ref, *, dims):
    (n_pt, d_pt, n_x, d_x, n_z, d_z, n_d, d_d, n_c, L1, L2, TB, D) = dims
    w1 = w1_ref[...]
    r0 = d_pt
    r1 = d_pt + d_x
    r2 = d_pt + d_x + d_z
    r3 = d_pt + d_x + d_z + d_d
    a_pt = jnp.dot(ept_ref[...], w1[0:r0, :],
                   preferred_element_type=jnp.float32)
    a_x = jnp.dot(ex_ref[...], w1[r0:r1, :],
                  preferred_element_type=jnp.float32)
    a_z = jnp.dot(ez_ref[...], w1[r1:r2, :],
                  preferred_element_type=jnp.float32)
    a_d = jnp.dot(ed_ref[...], w1[r2:r3, :],
                  preferred_element_type=jnp.float32)
    a_c = w1[r3:r3 + n_c, :] * istd_ref[...]
    z4 = jnp.zeros((n_c, D), jnp.float32)
    a1_ref[...] = jnp.concatenate(
        [z4, a_c, a_pt, jnp.zeros((L1 - 2 * n_c - n_pt, D), jnp.float32)],
        axis=0)
    pad2 = jnp.zeros((L2 - n_x - n_z - n_d, D), jnp.float32)
    a2_ref[...] = jnp.concatenate([a_x, a_z, a_d, pad2], axis=0)


def _encoder_kernel(pk_ref, a1_ref, a2_ref, b1_ref, w2_ref, b2_ref,
                    moff_ref, out_ref, *, dims):
    (n_pt, d_pt, n_x, d_x, n_z, d_z, n_d, d_d, n_c, L1, L2, TB, D) = dims

    pk = pk_ref[...]
    iota1 = jax.lax.broadcasted_iota(jnp.int32, (TB, L1), 1)
    pkm = pk * moff_ref[0:1, 0:8] + moff_ref[1:2, 0:8]
    aug1 = jnp.pad(pkm, ((0, 0), (0, L1 - 2 * n_c)))
    aug1 += (iota1 == pk[:, 0:1].astype(jnp.int32) + 2 * n_c).astype(
        jnp.float32)

    iota2 = jax.lax.broadcasted_iota(jnp.int32, (TB, L2), 1)
    ix = pk[:, 1:2].astype(jnp.int32)
    iz = pk[:, 2:3].astype(jnp.int32)
    id_ = pk[:, 3:4].astype(jnp.int32)
    aug2 = (iota2 == ix).astype(jnp.float32)
    aug2 += (iota2 == iz + n_x).astype(jnp.float32)
    aug2 += (iota2 == id_ + n_x + n_z).astype(jnp.float32)

    h = jnp.dot(aug1, a1_ref[...], preferred_element_type=jnp.float32)
    h += jnp.dot(aug2, a2_ref[...], preferred_element_type=jnp.float32)
    h += b1_ref[...]
    h = 0.5 * h * (1.0 + jax.lax.erf(h * 0.7071067811865476))
    out = jnp.dot(h, w2_ref[...], preferred_element_type=jnp.float32)
    out_ref[...] = out + b2_ref[...]


@jax.jit
def kernel(pitch_type_id, x_bin, z_bin, description_id, cont,
           emb_pitch_type, emb_x, emb_z, emb_desc,
           W1, b1, W2, b2, cont_mean, cont_std):
    B, S = pitch_type_id.shape
    T = B * S
    n_pt, d_pt = emb_pitch_type.shape
    n_x, d_x = emb_x.shape
    n_z, d_z = emb_z.shape
    n_d, d_d = emb_desc.shape
    n_c = cont.shape[-1]
    D = W2.shape[1]
    L1 = _round_up(2 * n_c + n_pt, 8)
    L2 = _round_up(n_x + n_z + n_d, 8)

    TB = 2048
    assert T % TB == 0
    G = T // TB

    istd = 1.0 / jnp.clip(cont_std, 1e-6, None)

    packed = jnp.concatenate(
        [jnp.stack([pitch_type_id, x_bin, z_bin, description_id],
                   axis=-1).astype(jnp.float32),
         cont.reshape(T, n_c).reshape(B, S, n_c)],
        axis=-1).reshape(T, 2 * n_c)

    mrow = jnp.concatenate([jnp.zeros((n_c,)), jnp.ones((n_c,))])
    orow = jnp.concatenate([jnp.zeros((n_c,)), -cont_mean * istd])
    moff = jnp.stack([mrow, orow]).astype(jnp.float32)

    dims = (n_pt, d_pt, n_x, d_x, n_z, d_z, n_d, d_d, n_c, L1, L2, TB, D)
    full = lambda shape: pl.BlockSpec(shape, lambda i: (0, 0))

    a1, a2 = pl.pallas_call(
        functools.partial(_fold_kernel, dims=dims),
        out_shape=(jax.ShapeDtypeStruct((L1, D), jnp.float32),
                   jax.ShapeDtypeStruct((L2, D), jnp.float32)),
    )(emb_pitch_type, emb_x, emb_z, emb_desc, W1, istd.reshape(n_c, 1))

    out = pl.pallas_call(
        functools.partial(_encoder_kernel, dims=dims),
        grid=(G,),
        in_specs=[
            pl.BlockSpec((TB, 2 * n_c), lambda i: (i, 0)),
            full((L1, D)),
            full((L2, D)),
            full((1, D)),
            full((D, D)),
            full((1, D)),
            full((2, 2 * n_c)),
        ],
        out_specs=pl.BlockSpec((TB, D), lambda i: (i, 0)),
        out_shape=jax.ShapeDtypeStruct((T, D), jnp.float32),
        compiler_params=pltpu.CompilerParams(
            dimension_semantics=("parallel",)),
    )(packed, a1, a2, b1.reshape(1, D), W2, b2.reshape(1, D), moff)

    return out.reshape(B, S, D)


# speedup vs baseline: 10.3157x; 1.0129x over previous
import functools

import jax
import jax.numpy as jnp
from jax.experimental import pallas as pl
from jax.experimental.pallas import tpu as pltpu


def _round_up(x, m):
    return (x + m - 1) // m * m


def _fold_kernel(ept_ref, ex_ref, ez_ref, ed_ref, w1_ref, istd_ref,
                 a1_ref, a2_ref, *, dims):
    (n_pt, d_pt, n_x, d_x, n_z, d_z, n_d, d_d, n_c, L1, L2, TB, D) = dims
    w1 = w1_ref[...]
    r0 = d_pt
    r1 = d_pt + d_x
    r2 = d_pt + d_x + d_z
    r3 = d_pt + d_x + d_z + d_d
    a_pt = jnp.dot(ept_ref[...], w1[0:r0, :],
                   preferred_element_type=jnp.float32)
    a_x = jnp.dot(ex_ref[...], w1[r0:r1, :],
                  preferred_element_type=jnp.float32)
    a_z = jnp.dot(ez_ref[...], w1[r1:r2, :],
                  preferred_element_type=jnp.float32)
    a_d = jnp.dot(ed_ref[...], w1[r2:r3, :],
                  preferred_element_type=jnp.float32)
    a_c = w1[r3:r3 + n_c, :] * istd_ref[...]
    z4 = jnp.zeros((n_c, D), jnp.float32)
    a1_ref[...] = jnp.concatenate(
        [z4, a_c, a_pt, jnp.zeros((L1 - 2 * n_c - n_pt, D), jnp.float32)],
        axis=0)
    pad2 = jnp.zeros((L2 - n_x - n_z - n_d, D), jnp.float32)
    a2_ref[...] = jnp.concatenate([a_x, a_z, a_d, pad2], axis=0)


def _encoder_kernel(pk_ref, a1_ref, a2_ref, b1_ref, w2_ref, b2_ref,
                    moff_ref, out_ref, *, dims):
    (n_pt, d_pt, n_x, d_x, n_z, d_z, n_d, d_d, n_c, L1, L2, TB, D) = dims

    pk = pk_ref[...]
    iota1 = jax.lax.broadcasted_iota(jnp.int32, (TB, L1), 1)
    pkm = pk * moff_ref[0:1, 0:8] + moff_ref[1:2, 0:8]
    aug1 = jnp.pad(pkm, ((0, 0), (0, L1 - 2 * n_c)))
    aug1 += (iota1 == pk[:, 0:1].astype(jnp.int32) + 2 * n_c).astype(
        jnp.float32)

    iota2 = jax.lax.broadcasted_iota(jnp.int32, (TB, L2), 1)
    ix = pk[:, 1:2].astype(jnp.int32)
    iz = pk[:, 2:3].astype(jnp.int32)
    id_ = pk[:, 3:4].astype(jnp.int32)
    aug2 = (iota2 == ix).astype(jnp.float32)
    aug2 += (iota2 == iz + n_x).astype(jnp.float32)
    aug2 += (iota2 == id_ + n_x + n_z).astype(jnp.float32)

    h = jnp.dot(aug1, a1_ref[...], preferred_element_type=jnp.float32)
    h += jnp.dot(aug2, a2_ref[...], preferred_element_type=jnp.float32)
    h += b1_ref[...]
    h = 0.5 * h * (1.0 + jax.lax.erf(h * 0.7071067811865476))
    out = jnp.dot(h, w2_ref[...], preferred_element_type=jnp.float32)
    out_ref[...] = out + b2_ref[...]


@jax.jit
def kernel(pitch_type_id, x_bin, z_bin, description_id, cont,
           emb_pitch_type, emb_x, emb_z, emb_desc,
           W1, b1, W2, b2, cont_mean, cont_std):
    B, S = pitch_type_id.shape
    T = B * S
    n_pt, d_pt = emb_pitch_type.shape
    n_x, d_x = emb_x.shape
    n_z, d_z = emb_z.shape
    n_d, d_d = emb_desc.shape
    n_c = cont.shape[-1]
    D = W2.shape[1]
    L1 = _round_up(2 * n_c + n_pt, 8)
    L2 = _round_up(n_x + n_z + n_d, 8)

    TB = 4096
    assert T % TB == 0
    G = T // TB

    istd = 1.0 / jnp.clip(cont_std, 1e-6, None)

    packed = jnp.concatenate(
        [jnp.stack([pitch_type_id, x_bin, z_bin, description_id],
                   axis=-1).astype(jnp.float32),
         cont.reshape(T, n_c).reshape(B, S, n_c)],
        axis=-1).reshape(T, 2 * n_c)

    mrow = jnp.concatenate([jnp.zeros((n_c,)), jnp.ones((n_c,))])
    orow = jnp.concatenate([jnp.zeros((n_c,)), -cont_mean * istd])
    moff = jnp.stack([mrow, orow]).astype(jnp.float32)

    dims = (n_pt, d_pt, n_x, d_x, n_z, d_z, n_d, d_d, n_c, L1, L2, TB, D)
    full = lambda shape: pl.BlockSpec(shape, lambda i: (0, 0))

    a1, a2 = pl.pallas_call(
        functools.partial(_fold_kernel, dims=dims),
        out_shape=(jax.ShapeDtypeStruct((L1, D), jnp.float32),
                   jax.ShapeDtypeStruct((L2, D), jnp.float32)),
    )(emb_pitch_type, emb_x, emb_z, emb_desc, W1, istd.reshape(n_c, 1))

    out = pl.pallas_call(
        functools.partial(_encoder_kernel, dims=dims),
        grid=(G,),
        in_specs=[
            pl.BlockSpec((TB, 2 * n_c), lambda i: (i, 0)),
            full((L1, D)),
            full((L2, D)),
            full((1, D)),
            full((D, D)),
            full((1, D)),
            full((2, 2 * n_c)),
        ],
        out_specs=pl.BlockSpec((TB, D), lambda i: (i, 0)),
        out_shape=jax.ShapeDtypeStruct((T, D), jnp.float32),
        compiler_params=pltpu.CompilerParams(
            dimension_semantics=("parallel",)),
    )(packed, a1, a2, b1.reshape(1, D), W2, b2.reshape(1, D), moff)

    return out.reshape(B, S, D)


# speedup vs baseline: 10.4499x; 1.0130x over previous
import functools

import jax
import jax.numpy as jnp
from jax.experimental import pallas as pl
from jax.experimental.pallas import tpu as pltpu


def _round_up(x, m):
    return (x + m - 1) // m * m


def _fold_kernel(ept_ref, ex_ref, ez_ref, ed_ref, w1_ref, istd_ref,
                 a1_ref, a2_ref, *, dims):
    (n_pt, d_pt, n_x, d_x, n_z, d_z, n_d, d_d, n_c, L1, L2, TB, D) = dims
    w1 = w1_ref[...]
    r0 = d_pt
    r1 = d_pt + d_x
    r2 = d_pt + d_x + d_z
    r3 = d_pt + d_x + d_z + d_d
    a_pt = jnp.dot(ept_ref[...], w1[0:r0, :],
                   preferred_element_type=jnp.float32)
    a_x = jnp.dot(ex_ref[...], w1[r0:r1, :],
                  preferred_element_type=jnp.float32)
    a_z = jnp.dot(ez_ref[...], w1[r1:r2, :],
                  preferred_element_type=jnp.float32)
    a_d = jnp.dot(ed_ref[...], w1[r2:r3, :],
                  preferred_element_type=jnp.float32)
    a_c = w1[r3:r3 + n_c, :] * istd_ref[...]
    z4 = jnp.zeros((n_c, D), jnp.float32)
    a1_ref[...] = jnp.concatenate(
        [z4, a_c, a_pt, jnp.zeros((L1 - 2 * n_c - n_pt, D), jnp.float32)],
        axis=0)
    pad2 = jnp.zeros((L2 - n_x - n_z - n_d, D), jnp.float32)
    a2_ref[...] = jnp.concatenate([a_x, a_z, a_d, pad2], axis=0)


def _encoder_kernel(pk_ref, a1_ref, a2_ref, b1_ref, w2_ref, b2_ref,
                    moff_ref, out_ref, *, dims):
    (n_pt, d_pt, n_x, d_x, n_z, d_z, n_d, d_d, n_c, L1, L2, TB, D) = dims

    pk = pk_ref[...]
    iota1 = jax.lax.broadcasted_iota(jnp.int32, (TB, L1), 1)
    pkm = pk * moff_ref[0:1, 0:8] + moff_ref[1:2, 0:8]
    aug1 = jnp.pad(pkm, ((0, 0), (0, L1 - 2 * n_c)))
    aug1 += (iota1 == pk[:, 0:1].astype(jnp.int32) + 2 * n_c).astype(
        jnp.float32)

    iota2 = jax.lax.broadcasted_iota(jnp.int32, (TB, L2), 1)
    ix = pk[:, 1:2].astype(jnp.int32)
    iz = pk[:, 2:3].astype(jnp.int32)
    id_ = pk[:, 3:4].astype(jnp.int32)
    aug2 = (iota2 == ix).astype(jnp.float32)
    aug2 += (iota2 == iz + n_x).astype(jnp.float32)
    aug2 += (iota2 == id_ + n_x + n_z).astype(jnp.float32)

    h = jnp.dot(aug1, a1_ref[...], preferred_element_type=jnp.float32)
    h += jnp.dot(aug2, a2_ref[...], preferred_element_type=jnp.float32)
    h += b1_ref[...]
    h = 0.5 * h * (1.0 + jax.lax.erf(h * 0.7071067811865476))
    out = jnp.dot(h, w2_ref[...], preferred_element_type=jnp.float32)
    out_ref[...] = out + b2_ref[...]


@jax.jit
def kernel(pitch_type_id, x_bin, z_bin, description_id, cont,
           emb_pitch_type, emb_x, emb_z, emb_desc,
           W1, b1, W2, b2, cont_mean, cont_std):
    B, S = pitch_type_id.shape
    T = B * S
    n_pt, d_pt = emb_pitch_type.shape
    n_x, d_x = emb_x.shape
    n_z, d_z = emb_z.shape
    n_d, d_d = emb_desc.shape
    n_c = cont.shape[-1]
    D = W2.shape[1]
    L1 = _round_up(2 * n_c + n_pt, 8)
    L2 = _round_up(n_x + n_z + n_d, 8)

    TB = 8192
    assert T % TB == 0
    G = T // TB

    istd = 1.0 / jnp.clip(cont_std, 1e-6, None)

    packed = jnp.concatenate(
        [jnp.stack([pitch_type_id, x_bin, z_bin, description_id],
                   axis=-1).astype(jnp.float32),
         cont.reshape(T, n_c).reshape(B, S, n_c)],
        axis=-1).reshape(T, 2 * n_c)

    mrow = jnp.concatenate([jnp.zeros((n_c,)), jnp.ones((n_c,))])
    orow = jnp.concatenate([jnp.zeros((n_c,)), -cont_mean * istd])
    moff = jnp.stack([mrow, orow]).astype(jnp.float32)

    dims = (n_pt, d_pt, n_x, d_x, n_z, d_z, n_d, d_d, n_c, L1, L2, TB, D)
    full = lambda shape: pl.BlockSpec(shape, lambda i: (0, 0))

    a1, a2 = pl.pallas_call(
        functools.partial(_fold_kernel, dims=dims),
        out_shape=(jax.ShapeDtypeStruct((L1, D), jnp.float32),
                   jax.ShapeDtypeStruct((L2, D), jnp.float32)),
    )(emb_pitch_type, emb_x, emb_z, emb_desc, W1, istd.reshape(n_c, 1))

    out = pl.pallas_call(
        functools.partial(_encoder_kernel, dims=dims),
        grid=(G,),
        in_specs=[
            pl.BlockSpec((TB, 2 * n_c), lambda i: (i, 0)),
            full((L1, D)),
            full((L2, D)),
            full((1, D)),
            full((D, D)),
            full((1, D)),
            full((2, 2 * n_c)),
        ],
        out_specs=pl.BlockSpec((TB, D), lambda i: (i, 0)),
        out_shape=jax.ShapeDtypeStruct((T, D), jnp.float32),
        compiler_params=pltpu.CompilerParams(
            dimension_semantics=("parallel",)),
    )(packed, a1, a2, b1.reshape(1, D), W2, b2.reshape(1, D), moff)

    return out.reshape(B, S, D)


# speedup vs baseline: 10.6171x; 1.0160x over previous
import functools

import jax
import jax.numpy as jnp
from jax.experimental import pallas as pl
from jax.experimental.pallas import tpu as pltpu


def _round_up(x, m):
    return (x + m - 1) // m * m


def _fold_kernel(ept_ref, ex_ref, ez_ref, ed_ref, w1_ref, istd_ref,
                 a1_ref, a2_ref, *, dims):
    (n_pt, d_pt, n_x, d_x, n_z, d_z, n_d, d_d, n_c, L1, L2, RB, D) = dims
    w1 = w1_ref[...]
    r0 = d_pt
    r1 = d_pt + d_x
    r2 = d_pt + d_x + d_z
    r3 = d_pt + d_x + d_z + d_d
    a_pt = jnp.dot(ept_ref[...], w1[0:r0, :],
                   preferred_element_type=jnp.float32)
    a_x = jnp.dot(ex_ref[...], w1[r0:r1, :],
                  preferred_element_type=jnp.float32)
    a_z = jnp.dot(ez_ref[...], w1[r1:r2, :],
                  preferred_element_type=jnp.float32)
    a_d = jnp.dot(ed_ref[...], w1[r2:r3, :],
                  preferred_element_type=jnp.float32)
    a_c = w1[r3:r3 + n_c, :] * istd_ref[...]
    parts1 = [a_pt, a_c]
    if L1 > n_pt + n_c:
        parts1.append(jnp.zeros((L1 - n_pt - n_c, D), jnp.float32))
    a1_ref[...] = jnp.concatenate(parts1, axis=0)
    pad2 = jnp.zeros((L2 - n_x - n_z - n_d, D), jnp.float32)
    a2_ref[...] = jnp.concatenate([a_x, a_z, a_d, pad2], axis=0)


def _encoder_kernel(pt_ref, x_ref, z_ref, d_ref, cont_ref,
                    a1_ref, a2_ref, b1_ref, w2_ref, b2_ref,
                    mean_ref, istd_ref, out_ref, *, dims):
    (n_pt, d_pt, n_x, d_x, n_z, d_z, n_d, d_d, n_c, L1, L2, RB, D) = dims
    S = pt_ref.shape[1]

    iota1 = jax.lax.broadcasted_iota(jnp.int32, (RB, S, n_pt), 2)
    oh_pt = (iota1 == pt_ref[...][:, :, None]).astype(jnp.float32)
    cont_n = (cont_ref[...] - mean_ref[0, 0, :]) * istd_ref[0, 0, :]
    aug1 = jnp.concatenate([oh_pt, cont_n], axis=2)

    iota2 = jax.lax.broadcasted_iota(jnp.int32, (RB, S, L2), 2)
    aug2 = (iota2 == x_ref[...][:, :, None]).astype(jnp.float32)
    aug2 += (iota2 == z_ref[...][:, :, None] + n_x).astype(jnp.float32)
    aug2 += (iota2 == d_ref[...][:, :, None] + n_x + n_z).astype(jnp.float32)

    dn = (((2,), (0,)), ((), ()))
    h = jax.lax.dot_general(aug1, a1_ref[...], dn,
                            preferred_element_type=jnp.float32)
    h += jax.lax.dot_general(aug2, a2_ref[...], dn,
                             preferred_element_type=jnp.float32)
    h += b1_ref[0, 0, :]
    h = 0.5 * h * (1.0 + jax.lax.erf(h * 0.7071067811865476))
    out = jax.lax.dot_general(h, w2_ref[...], dn,
                              preferred_element_type=jnp.float32)
    out_ref[...] = out + b2_ref[0, 0, :]


@jax.jit
def kernel(pitch_type_id, x_bin, z_bin, description_id, cont,
           emb_pitch_type, emb_x, emb_z, emb_desc,
           W1, b1, W2, b2, cont_mean, cont_std):
    B, S = pitch_type_id.shape
    n_pt, d_pt = emb_pitch_type.shape
    n_x, d_x = emb_x.shape
    n_z, d_z = emb_z.shape
    n_d, d_d = emb_desc.shape
    n_c = cont.shape[-1]
    D = W2.shape[1]
    L1 = _round_up(n_pt + n_c, 8)
    L2 = _round_up(n_x + n_z + n_d, 8)

    RB = 16
    assert B % RB == 0
    G = B // RB

    istd = 1.0 / jnp.clip(cont_std, 1e-6, None)
    dims = (n_pt, d_pt, n_x, d_x, n_z, d_z, n_d, d_d, n_c, L1, L2, RB, D)
    full = lambda shape: pl.BlockSpec(shape, lambda i: tuple(0 for _ in shape))

    a1, a2 = pl.pallas_call(
        functools.partial(_fold_kernel, dims=dims),
        out_shape=(jax.ShapeDtypeStruct((L1, D), jnp.float32),
                   jax.ShapeDtypeStruct((L2, D), jnp.float32)),
    )(emb_pitch_type, emb_x, emb_z, emb_desc, W1, istd.reshape(n_c, 1))

    out = pl.pallas_call(
        functools.partial(_encoder_kernel, dims=dims),
        grid=(G,),
        in_specs=[
            pl.BlockSpec((RB, S), lambda i: (i, 0)),
            pl.BlockSpec((RB, S), lambda i: (i, 0)),
            pl.BlockSpec((RB, S), lambda i: (i, 0)),
            pl.BlockSpec((RB, S), lambda i: (i, 0)),
            pl.BlockSpec((RB, S, n_c), lambda i: (i, 0, 0)),
            full((L1, D)),
            full((L2, D)),
            pl.BlockSpec((1, 1, D), lambda i: (0, 0, 0)),
            full((D, D)),
            pl.BlockSpec((1, 1, D), lambda i: (0, 0, 0)),
            pl.BlockSpec((1, 1, n_c), lambda i: (0, 0, 0)),
            pl.BlockSpec((1, 1, n_c), lambda i: (0, 0, 0)),
        ],
        out_specs=pl.BlockSpec((RB, S, D), lambda i: (i, 0, 0)),
        out_shape=jax.ShapeDtypeStruct((B, S, D), jnp.float32),
        compiler_params=pltpu.CompilerParams(
            dimension_semantics=("parallel",)),
    )(pitch_type_id.astype(jnp.int32), x_bin.astype(jnp.int32),
      z_bin.astype(jnp.int32), description_id.astype(jnp.int32), cont,
      a1, a2, b1.reshape(1, 1, D), W2, b2.reshape(1, 1, D),
      cont_mean.reshape(1, 1, n_c), istd.reshape(1, 1, n_c))

    return out


# speedup vs baseline: 11.2433x; 1.0590x over previous
import functools

import jax
import jax.numpy as jnp
from jax.experimental import pallas as pl
from jax.experimental.pallas import tpu as pltpu


def _round_up(x, m):
    return (x + m - 1) // m * m


def _fold_kernel(ept_ref, ex_ref, ez_ref, ed_ref, w1_ref, istd_ref, b1_ref,
                 a1_ref, a2_ref, *, dims):
    (n_pt, d_pt, n_x, d_x, n_z, d_z, n_d, d_d, n_c, L1, L2, RB, D) = dims
    w1 = w1_ref[...]
    r0 = d_pt
    r1 = d_pt + d_x
    r2 = d_pt + d_x + d_z
    r3 = d_pt + d_x + d_z + d_d
    a_pt = jnp.dot(ept_ref[...], w1[0:r0, :],
                   preferred_element_type=jnp.float32) + b1_ref[...]
    a_x = jnp.dot(ex_ref[...], w1[r0:r1, :],
                  preferred_element_type=jnp.float32)
    a_z = jnp.dot(ez_ref[...], w1[r1:r2, :],
                  preferred_element_type=jnp.float32)
    a_d = jnp.dot(ed_ref[...], w1[r2:r3, :],
                  preferred_element_type=jnp.float32)
    a_c = w1[r3:r3 + n_c, :] * istd_ref[...]
    parts1 = [a_pt, a_c]
    if L1 > n_pt + n_c:
        parts1.append(jnp.zeros((L1 - n_pt - n_c, D), jnp.float32))
    a1_ref[...] = jnp.concatenate(parts1, axis=0)
    pad2 = jnp.zeros((L2 - n_x - n_z - n_d, D), jnp.float32)
    a2_ref[...] = jnp.concatenate([a_x, a_z, a_d, pad2], axis=0)


def _encoder_kernel(pt_ref, x_ref, z_ref, d_ref, cont_ref,
                    a1_ref, a2_ref, w2_ref, b2_ref,
                    mean_ref, istd_ref, out_ref, *, dims):
    (n_pt, d_pt, n_x, d_x, n_z, d_z, n_d, d_d, n_c, L1, L2, RB, D) = dims
    S = pt_ref.shape[1]

    iota1 = jax.lax.broadcasted_iota(jnp.int32, (RB, S, n_pt), 2)
    oh_pt = (iota1 == pt_ref[...][:, :, None]).astype(jnp.float32)
    cont_n = (cont_ref[...] - mean_ref[0, 0, :]) * istd_ref[0, 0, :]
    aug1 = jnp.concatenate([oh_pt, cont_n], axis=2)

    iota2 = jax.lax.broadcasted_iota(jnp.int32, (RB, S, L2), 2)
    aug2 = (iota2 == x_ref[...][:, :, None]).astype(jnp.float32)
    aug2 += (iota2 == z_ref[...][:, :, None] + n_x).astype(jnp.float32)
    aug2 += (iota2 == d_ref[...][:, :, None] + n_x + n_z).astype(jnp.float32)

    dn = (((2,), (0,)), ((), ()))
    h = jax.lax.dot_general(aug1, a1_ref[...], dn,
                            preferred_element_type=jnp.float32)
    h += jax.lax.dot_general(aug2, a2_ref[...], dn,
                             preferred_element_type=jnp.float32)
    h = 0.5 * h * (1.0 + jax.lax.erf(h * 0.7071067811865476))
    out = jax.lax.dot_general(h, w2_ref[...], dn,
                              preferred_element_type=jnp.float32)
    out_ref[...] = out + b2_ref[0, 0, :]


@jax.jit
def kernel(pitch_type_id, x_bin, z_bin, description_id, cont,
           emb_pitch_type, emb_x, emb_z, emb_desc,
           W1, b1, W2, b2, cont_mean, cont_std):
    B, S = pitch_type_id.shape
    n_pt, d_pt = emb_pitch_type.shape
    n_x, d_x = emb_x.shape
    n_z, d_z = emb_z.shape
    n_d, d_d = emb_desc.shape
    n_c = cont.shape[-1]
    D = W2.shape[1]
    L1 = _round_up(n_pt + n_c, 8)
    L2 = _round_up(n_x + n_z + n_d, 8)

    RB = 32
    assert B % RB == 0
    G = B // RB

    istd = 1.0 / jnp.clip(cont_std, 1e-6, None)
    dims = (n_pt, d_pt, n_x, d_x, n_z, d_z, n_d, d_d, n_c, L1, L2, RB, D)
    full = lambda shape: pl.BlockSpec(shape, lambda i: tuple(0 for _ in shape))

    a1, a2 = pl.pallas_call(
        functools.partial(_fold_kernel, dims=dims),
        out_shape=(jax.ShapeDtypeStruct((L1, D), jnp.float32),
                   jax.ShapeDtypeStruct((L2, D), jnp.float32)),
    )(emb_pitch_type, emb_x, emb_z, emb_desc, W1, istd.reshape(n_c, 1),
      b1.reshape(1, D))

    out = pl.pallas_call(
        functools.partial(_encoder_kernel, dims=dims),
        grid=(G,),
        in_specs=[
            pl.BlockSpec((RB, S), lambda i: (i, 0)),
            pl.BlockSpec((RB, S), lambda i: (i, 0)),
            pl.BlockSpec((RB, S), lambda i: (i, 0)),
            pl.BlockSpec((RB, S), lambda i: (i, 0)),
            pl.BlockSpec((RB, S, n_c), lambda i: (i, 0, 0)),
            full((L1, D)),
            full((L2, D)),
            full((D, D)),
            pl.BlockSpec((1, 1, D), lambda i: (0, 0, 0)),
            pl.BlockSpec((1, 1, n_c), lambda i: (0, 0, 0)),
            pl.BlockSpec((1, 1, n_c), lambda i: (0, 0, 0)),
        ],
        out_specs=pl.BlockSpec((RB, S, D), lambda i: (i, 0, 0)),
        out_shape=jax.ShapeDtypeStruct((B, S, D), jnp.float32),
        compiler_params=pltpu.CompilerParams(
            dimension_semantics=("parallel",)),
    )(pitch_type_id.astype(jnp.int32), x_bin.astype(jnp.int32),
      z_bin.astype(jnp.int32), description_id.astype(jnp.int32), cont,
      a1, a2, W2, b2.reshape(1, 1, D),
      cont_mean.reshape(1, 1, n_c), istd.reshape(1, 1, n_c))

    return out
